# Initial kernel scaffold; baseline (speedup 1.0000x reference)
import jax
import jax.numpy as jnp
from jax.experimental import pallas as pl


def kernel(inputs):
    raise NotImplementedError("write your pallas kernel here")


# speedup vs baseline: 1.1023x; 1.1023x over previous
import jax
import jax.numpy as jnp
import numpy as np
from jax import lax
from jax.experimental import pallas as pl

B, C, H, W = 4, 96, 224, 224
OH, OW = H // 2, W // 2
L = OH * OW
BC = B * C

ROWS = 8
GRID = BC // ROWS

_KS1 = 42
_KS2 = 0x1BD11BDA ^ 42
_TINY = np.float32(np.finfo(np.float32).tiny)


def _rotl(x, r):
    return (x << r) | lax.shift_right_logical(x, 32 - r)


def _rounds(x0, x1, rots):
    for r in rots:
        x0 = x0 + x1
        x1 = x0 ^ _rotl(x1, r)
    return x0, x1


def _threefry2x32(x0, x1):
    rot_a = (13, 15, 26, 6)
    rot_b = (17, 29, 16, 24)
    x1 = x1 + _KS1
    x0, x1 = _rounds(x0, x1, rot_a)
    x0, x1 = x0 + _KS1, x1 + (_KS2 + 1)
    x0, x1 = _rounds(x0, x1, rot_b)
    x0, x1 = x0 + _KS2, x1 + 2
    x0, x1 = _rounds(x0, x1, rot_a)
    x0, x1 = x0, x1 + (_KS1 + 3)
    x0, x1 = _rounds(x0, x1, rot_b)
    x0, x1 = x0 + _KS1, x1 + (_KS2 + 4)
    x0, x1 = _rounds(x0, x1, rot_a)
    x0, x1 = x0 + _KS2, x1 + 5
    return x0, x1


def _gumbel(bits):
    mant = lax.shift_right_logical(bits, 9) | 0x3F800000
    f = lax.bitcast_convert_type(mant, jnp.float32)
    u = jnp.maximum(_TINY, (f - 1.0) * (np.float32(1.0) - _TINY) + _TINY)
    return -jnp.log(-jnp.log(u))


def _pool_kernel(v_ref, out_ref):
    i = pl.program_id(0)
    bc = jax.lax.broadcasted_iota(jnp.int32, (ROWS, L), 0) + i * ROWS
    l = jax.lax.broadcasted_iota(jnp.int32, (ROWS, L), 1)
    base = (bc * L + l) * 4
    bs = bv = None
    for j in range(4):
        o0, o1 = _threefry2x32(jnp.int32(0), base + j)
        v = v_ref[j]
        s = v + _gumbel(o0 ^ o1)
        if j == 0:
            bs, bv = s, v
        else:
            c = s > bs
            bs = jnp.where(c, s, bs)
            bv = jnp.where(c, v, bv)
    out_ref[...] = bv


def kernel(inputs):
    x = inputs.reshape(BC, OH, 2, OW, 2)
    v = jnp.transpose(x, (2, 4, 0, 1, 3)).reshape(4, BC, L)
    out = pl.pallas_call(
        _pool_kernel,
        grid=(GRID,),
        in_specs=[pl.BlockSpec((4, ROWS, L), lambda i: (0, i, 0))],
        out_specs=pl.BlockSpec((ROWS, L), lambda i: (i, 0)),
        out_shape=jax.ShapeDtypeStruct((BC, L), jnp.float32),
    )(v)
    return out.reshape(B, C, OH, OW)


# speedup vs baseline: 1.5663x; 1.4209x over previous
import jax
import jax.numpy as jnp
import numpy as np
from jax import lax
from jax.experimental import pallas as pl
from jax.experimental.pallas import tpu as pltpu

B, C, H, W = 4, 96, 224, 224
OH, OW = H // 2, W // 2
L = OH * OW
BC = B * C

ROWS = 8
GRID = BC // ROWS

_KS1 = 42
_KS2 = 0x1BD11BDA ^ 42
_TINY = np.float32(np.finfo(np.float32).tiny)


def _rotl(x, r):
    return (x << r) | lax.shift_right_logical(x, 32 - r)


def _rounds(x0, x1, rots):
    for r in rots:
        x0 = x0 + x1
        x1 = x0 ^ _rotl(x1, r)
    return x0, x1


def _threefry2x32(x0, x1):
    rot_a = (13, 15, 26, 6)
    rot_b = (17, 29, 16, 24)
    x1 = x1 + _KS1
    x0, x1 = _rounds(x0, x1, rot_a)
    x0, x1 = x0 + _KS1, x1 + (_KS2 + 1)
    x0, x1 = _rounds(x0, x1, rot_b)
    x0, x1 = x0 + _KS2, x1 + 2
    x0, x1 = _rounds(x0, x1, rot_a)
    x0, x1 = x0, x1 + (_KS1 + 3)
    x0, x1 = _rounds(x0, x1, rot_b)
    x0, x1 = x0 + _KS1, x1 + (_KS2 + 4)
    x0, x1 = _rounds(x0, x1, rot_a)
    x0, x1 = x0 + _KS2, x1 + 5
    return x0, x1


def _gumbel(cnt):
    o0, o1 = _threefry2x32(jnp.int32(0), cnt)
    bits = o0 ^ o1
    mant = lax.shift_right_logical(bits, 9) | 0x3F800000
    f = lax.bitcast_convert_type(mant, jnp.float32)
    u = jnp.maximum(f - 1.0, _TINY)
    return -jnp.log(-jnp.log(u))


def _pool_kernel(x_ref, out_ref):
    i = pl.program_id(0)
    sh = (ROWS, OH, W)
    bc = lax.broadcasted_iota(jnp.int32, sh, 0) + i * ROWS
    oh = lax.broadcasted_iota(jnp.int32, sh, 1)
    cc = lax.broadcasted_iota(jnp.int32, sh, 2)
    base = ((bc * OH + oh) * OW + lax.shift_right_logical(cc, 1)) * 4 + (cc & 1)
    v0 = x_ref[:, :, 0, :]
    v1 = x_ref[:, :, 1, :]
    s0 = v0 + _gumbel(base)
    s1 = v1 + _gumbel(base + 2)
    cr = s1 > s0
    sr = jnp.where(cr, s1, s0)
    vr = jnp.where(cr, v1, v0)
    ss = pltpu.roll(sr, W - 1, 2)
    vs = pltpu.roll(vr, W - 1, 2)
    c = ss > sr
    win = jnp.where(c, vs, vr)
    for b in range(7):
        mask = (cc & (1 << b)) != 0
        win = jnp.where(mask, pltpu.roll(win, W - (1 << b), 2), win)
    out_ref[...] = win[:, :, :OW]


def kernel(inputs):
    x = inputs.reshape(BC, OH, 2, W)
    out = pl.pallas_call(
        _pool_kernel,
        grid=(GRID,),
        in_specs=[pl.BlockSpec((ROWS, OH, 2, W), lambda i: (i, 0, 0, 0))],
        out_specs=pl.BlockSpec((ROWS, OH, OW), lambda i: (i, 0, 0)),
        out_shape=jax.ShapeDtypeStruct((BC, OH, OW), jnp.float32),
    )(x)
    return out.reshape(B, C, OH, OW)
